# baseline (speedup 1.0000x reference)
import jax
import jax.numpy as jnp
from jax.experimental import pallas as pl

_BLK = 256


def _add_body(x_ref, pos_ref, o_ref):
    o_ref[...] = x_ref[...] + pos_ref[...]


def kernel(x, pos_table):
    B, S, D = x.shape
    pos = pos_table[:S][None]
    grid = (S // _BLK, B)
    return pl.pallas_call(
        _add_body,
        grid=grid,
        in_specs=[
            pl.BlockSpec((1, _BLK, D), lambda i, b: (b, i, 0)),
            pl.BlockSpec((1, _BLK, D), lambda i, b: (0, i, 0)),
        ],
        out_specs=pl.BlockSpec((1, _BLK, D), lambda i, b: (b, i, 0)),
        out_shape=jax.ShapeDtypeStruct((B, S, D), x.dtype),
    )(x, pos)


# speedup vs baseline: 1.3538x; 1.3538x over previous
import jax
import jax.numpy as jnp
from jax.experimental import pallas as pl

_BLK = 1024


def _add_body(x_ref, pos_ref, o_ref):
    o_ref[...] = x_ref[...] + pos_ref[...]


def kernel(x, pos_table):
    B, S, D = x.shape
    pos = pos_table[:S][None]
    grid = (S // _BLK, B)
    return pl.pallas_call(
        _add_body,
        grid=grid,
        in_specs=[
            pl.BlockSpec((1, _BLK, D), lambda i, b: (b, i, 0)),
            pl.BlockSpec((1, _BLK, D), lambda i, b: (0, i, 0)),
        ],
        out_specs=pl.BlockSpec((1, _BLK, D), lambda i, b: (b, i, 0)),
        out_shape=jax.ShapeDtypeStruct((B, S, D), x.dtype),
    )(x, pos)


# speedup vs baseline: 1.4250x; 1.0526x over previous
import jax
import jax.numpy as jnp
from jax.experimental import pallas as pl

_BLK = 2048


def _add_body(x_ref, pos_ref, o_ref):
    o_ref[...] = x_ref[...] + pos_ref[...]


def kernel(x, pos_table):
    B, S, D = x.shape
    pos = pos_table[:S][None]
    grid = (S // _BLK, B)
    return pl.pallas_call(
        _add_body,
        grid=grid,
        in_specs=[
            pl.BlockSpec((1, _BLK, D), lambda i, b: (b, i, 0)),
            pl.BlockSpec((1, _BLK, D), lambda i, b: (0, i, 0)),
        ],
        out_specs=pl.BlockSpec((1, _BLK, D), lambda i, b: (b, i, 0)),
        out_shape=jax.ShapeDtypeStruct((B, S, D), x.dtype),
    )(x, pos)
